# baseline (speedup 1.0000x reference)
import jax
import jax.numpy as jnp
from jax.experimental import pallas as pl
from jax.experimental.pallas import tpu as pltpu

R = 128
V = 100000
C = 5000
NSTEP = V // C


def _body(p_ref, o_ref, best_val, best_idx):
    ph = pl.program_id(0)
    j = pl.program_id(1)

    @pl.when(ph == 0)
    def _accumulate():
        @pl.when(j == 0)
        def _init():
            best_val[...] = jnp.full((1, R), -jnp.inf, jnp.float32)
            best_idx[...] = jnp.zeros((1, R), jnp.int32)

        x = p_ref[...]
        m = jnp.max(x, axis=0, keepdims=True)
        row = jax.lax.broadcasted_iota(jnp.int32, (C, R), 0)
        cand = jnp.where(x == m, row, jnp.int32(2**31 - 1))
        first = jnp.min(cand, axis=0, keepdims=True) + j * C
        upd = m > best_val[...]
        best_idx[...] = jnp.where(upd, first, best_idx[...])
        best_val[...] = jnp.where(upd, m, best_val[...])

    @pl.when(ph == 1)
    def _write():
        row = jax.lax.broadcasted_iota(jnp.int32, (C, R), 0) + j * C
        o_ref[...] = (row == best_idx[...]).astype(jnp.float32)


_onehot_argmax_t = pl.pallas_call(
    _body,
    grid=(2, NSTEP),
    in_specs=[pl.BlockSpec(
        (C, R), lambda p, j: (j * (1 - p) + (NSTEP - 1) * p, 0))],
    out_specs=pl.BlockSpec((C, R), lambda p, j: (j * p, 0)),
    out_shape=jax.ShapeDtypeStruct((V, R), jnp.float32),
    scratch_shapes=[
        pltpu.VMEM((1, R), jnp.float32),
        pltpu.VMEM((1, R), jnp.int32),
    ],
)


def kernel(probs):
    return _onehot_argmax_t(probs.T).T


# speedup vs baseline: 1.1144x; 1.1144x over previous
import jax
import jax.numpy as jnp
from jax.experimental import pallas as pl
from jax.experimental.pallas import tpu as pltpu

R = 128
V = 100000
C = 10000
NSTEP = V // C


def _body(p_ref, o_ref, best_val, best_idx):
    ph = pl.program_id(0)
    j = pl.program_id(1)

    @pl.when(ph == 0)
    def _accumulate():
        @pl.when(j == 0)
        def _init():
            best_val[...] = jnp.full((1, R), -jnp.inf, jnp.float32)
            best_idx[...] = jnp.zeros((1, R), jnp.int32)

        x = p_ref[...]
        m = jnp.max(x, axis=0, keepdims=True)
        row = jax.lax.broadcasted_iota(jnp.int32, (C, R), 0)
        cand = jnp.where(x == m, row, jnp.int32(2**31 - 1))
        first = jnp.min(cand, axis=0, keepdims=True) + j * C
        upd = m > best_val[...]
        best_idx[...] = jnp.where(upd, first, best_idx[...])
        best_val[...] = jnp.where(upd, m, best_val[...])

    @pl.when(ph == 1)
    def _write():
        row = jax.lax.broadcasted_iota(jnp.int32, (C, R), 0) + j * C
        o_ref[...] = (row == best_idx[...]).astype(jnp.float32)


_onehot_argmax_t = pl.pallas_call(
    _body,
    grid=(2, NSTEP),
    in_specs=[pl.BlockSpec(
        (C, R), lambda p, j: (j * (1 - p) + (NSTEP - 1) * p, 0))],
    out_specs=pl.BlockSpec((C, R), lambda p, j: (j * p, 0)),
    out_shape=jax.ShapeDtypeStruct((V, R), jnp.float32),
    scratch_shapes=[
        pltpu.VMEM((1, R), jnp.float32),
        pltpu.VMEM((1, R), jnp.int32),
    ],
)


def kernel(probs):
    return _onehot_argmax_t(probs.T).T
